# Initial kernel scaffold; baseline (speedup 1.0000x reference)
import functools

import jax
import jax.numpy as jnp
from jax import lax
from jax.experimental import pallas as pl
from jax.experimental.pallas import tpu as pltpu
from jax.experimental.pallas import tpu_sc as plsc

N_NODES = 10000
D = 128
D_OUT = 64
G = 64

NC, NS = 2, 16
NW = NC * NS
CHUNK = 128
N_PAD = 10240
ROWS_PT = N_PAD // NS
BLK = 1024


def _mesh():
    return plsc.VectorSubcoreMesh(core_axis_name="c", subcore_axis_name="s")


def _sc_degree(dstp, z16, ones16):
    n_chunks = dstp.shape[1]

    @functools.partial(
        pl.kernel,
        out_type=jax.ShapeDtypeStruct((NC * N_PAD, 16), jnp.float32),
        mesh=_mesh(),
        scratch_types=[
            pltpu.VMEM((n_chunks, CHUNK), jnp.int32),
            pltpu.VMEM((CHUNK, 16), jnp.float32),
            pltpu.VMEM_SHARED((N_PAD, 16), jnp.float32),
            pltpu.SemaphoreType.DMA,
        ],
    )
    def deg_kernel(dstp_hbm, z_hbm, ones_hbm, out_hbm, dst_v, ones_v, acc_sh, sem):
        c = lax.axis_index("c")
        s = lax.axis_index("s")
        wid = c * NS + s
        pltpu.sync_copy(dstp_hbm.at[wid], dst_v)
        pltpu.sync_copy(ones_hbm, ones_v)
        pltpu.sync_copy(z_hbm, acc_sh.at[pl.ds(s * ROWS_PT, ROWS_PT)])
        plsc.subcore_barrier()

        @pl.loop(0, n_chunks)
        def _(j):
            pltpu.sync_copy(ones_v, acc_sh.at[dst_v.at[j]], add=True)

        plsc.subcore_barrier()
        pltpu.sync_copy(
            acc_sh.at[pl.ds(s * ROWS_PT, ROWS_PT)],
            out_hbm.at[pl.ds(c * N_PAD + s * ROWS_PT, ROWS_PT)],
        )

    return deg_kernel(dstp, z16, ones16)


def _sc_aggregate(sfeat, srcp, dstp, z128):
    n_chunks = srcp.shape[1]

    @functools.partial(
        pl.kernel,
        out_type=jax.ShapeDtypeStruct((NC * N_PAD, D), jnp.float32),
        mesh=_mesh(),
        scratch_types=[
            pltpu.VMEM((n_chunks, CHUNK), jnp.int32),
            pltpu.VMEM((n_chunks, CHUNK), jnp.int32),
            pltpu.VMEM((CHUNK, D), jnp.float32),
            pltpu.VMEM((CHUNK, D), jnp.float32),
            pltpu.VMEM_SHARED((N_PAD, D), jnp.float32),
            pltpu.SemaphoreType.DMA,
            pltpu.SemaphoreType.DMA,
        ],
    )
    def agg_kernel(s_hbm, srcp_hbm, dstp_hbm, z_hbm, out_hbm,
                   src_v, dst_v, rows_a, rows_b, acc_sh, sem_a, sem_b):
        c = lax.axis_index("c")
        s = lax.axis_index("s")
        wid = c * NS + s
        pltpu.sync_copy(srcp_hbm.at[wid], src_v)
        pltpu.sync_copy(dstp_hbm.at[wid], dst_v)
        pltpu.sync_copy(z_hbm, acc_sh.at[pl.ds(s * ROWS_PT, ROWS_PT)])
        plsc.subcore_barrier()

        pltpu.async_copy(s_hbm.at[src_v.at[0]], rows_a, sem_a)

        @pl.loop(0, n_chunks, step=2)
        def _(j):
            pltpu.async_copy(s_hbm.at[src_v.at[j + 1]], rows_b, sem_b)
            pltpu.make_async_copy(s_hbm.at[src_v.at[j]], rows_a, sem_a).wait()
            pltpu.sync_copy(rows_a, acc_sh.at[dst_v.at[j]], add=True)

            @pl.when(j + 2 < n_chunks)
            def _():
                pltpu.async_copy(s_hbm.at[src_v.at[j + 2]], rows_a, sem_a)

            pltpu.make_async_copy(s_hbm.at[src_v.at[j + 1]], rows_b, sem_b).wait()
            pltpu.sync_copy(rows_b, acc_sh.at[dst_v.at[j + 1]], add=True)

        plsc.subcore_barrier()
        pltpu.sync_copy(
            acc_sh.at[pl.ds(s * ROWS_PT, ROWS_PT)],
            out_hbm.at[pl.ds(c * N_PAD + s * ROWS_PT, ROWS_PT)],
        )

    return agg_kernel(sfeat, srcp, dstp, z128)


def _tc_matmul(x, w):
    def body(x_ref, w_ref, o_ref):
        o_ref[...] = jnp.dot(x_ref[...], w_ref[...],
                             preferred_element_type=jnp.float32)

    return pl.pallas_call(
        body,
        grid=(N_PAD // BLK,),
        in_specs=[
            pl.BlockSpec((BLK, D), lambda i: (i, 0)),
            pl.BlockSpec((D, D), lambda i: (0, 0)),
        ],
        out_specs=pl.BlockSpec((BLK, D), lambda i: (i, 0)),
        out_shape=jax.ShapeDtypeStruct((N_PAD, D), jnp.float32),
    )(x, w)


def _dis(d0_ref, d1_ref):
    return lax.rsqrt(d0_ref[:, :1] + d1_ref[:, :1] + 1.0)


def _tc_scale(xw, d0, d1):
    def body(xw_ref, d0_ref, d1_ref, o_ref):
        o_ref[...] = xw_ref[...] * _dis(d0_ref, d1_ref)

    return pl.pallas_call(
        body,
        grid=(N_PAD // BLK,),
        in_specs=[
            pl.BlockSpec((BLK, D), lambda i: (i, 0)),
            pl.BlockSpec((BLK, 16), lambda i: (i, 0)),
            pl.BlockSpec((BLK, 16), lambda i: (i, 0)),
        ],
        out_specs=pl.BlockSpec((BLK, D), lambda i: (i, 0)),
        out_shape=jax.ShapeDtypeStruct((N_PAD, D), jnp.float32),
    )(xw, d0, d1)


def _tc_mid(a0, a1, s1, d0, d1, b1, w2):
    def body(a0_ref, a1_ref, s1_ref, d0_ref, d1_ref, b1_ref, w2_ref, o_ref):
        dis = _dis(d0_ref, d1_ref)
        h = jnp.maximum((a0_ref[...] + a1_ref[...] + s1_ref[...]) * dis
                        + b1_ref[...], 0.0)
        o_ref[...] = jnp.dot(h, w2_ref[...],
                             preferred_element_type=jnp.float32) * dis

    return pl.pallas_call(
        body,
        grid=(N_PAD // BLK,),
        in_specs=[
            pl.BlockSpec((BLK, D), lambda i: (i, 0)),
            pl.BlockSpec((BLK, D), lambda i: (i, 0)),
            pl.BlockSpec((BLK, D), lambda i: (i, 0)),
            pl.BlockSpec((BLK, 16), lambda i: (i, 0)),
            pl.BlockSpec((BLK, 16), lambda i: (i, 0)),
            pl.BlockSpec((1, D), lambda i: (0, 0)),
            pl.BlockSpec((D, D), lambda i: (0, 0)),
        ],
        out_specs=pl.BlockSpec((BLK, D), lambda i: (i, 0)),
        out_shape=jax.ShapeDtypeStruct((N_PAD, D), jnp.float32),
    )(a0, a1, s1, d0, d1, b1, w2)


def _tc_final(q0, q1, s2, d0, d1, b2, batch3, wlin, blin):
    def body(q0_ref, q1_ref, s2_ref, d0_ref, d1_ref, b2_ref, bt_ref,
             wl_ref, bl_ref, o_ref, acc):
        i = pl.program_id(0)
        dis = _dis(d0_ref, d1_ref)
        h2 = jnp.maximum((q0_ref[...] + q1_ref[...] + s2_ref[...]) * dis
                         + b2_ref[...], 0.0)
        bcol = bt_ref[...].reshape(BLK, 1)
        onehot = (bcol == lax.broadcasted_iota(jnp.int32, (BLK, G), 1)
                  ).astype(jnp.float32)
        contrib = lax.dot_general(onehot, h2, (((0,), (0,)), ((), ())),
                                  preferred_element_type=jnp.float32)

        @pl.when(i == 0)
        def _():
            acc[...] = jnp.zeros_like(acc)

        acc[...] += contrib

        @pl.when(i == pl.num_programs(0) - 1)
        def _():
            o_ref[...] = jnp.dot(acc[...], wl_ref[...],
                                 preferred_element_type=jnp.float32) + bl_ref[...]

    return pl.pallas_call(
        body,
        grid=(N_PAD // BLK,),
        in_specs=[
            pl.BlockSpec((BLK, D), lambda i: (i, 0)),
            pl.BlockSpec((BLK, D), lambda i: (i, 0)),
            pl.BlockSpec((BLK, D), lambda i: (i, 0)),
            pl.BlockSpec((BLK, 16), lambda i: (i, 0)),
            pl.BlockSpec((BLK, 16), lambda i: (i, 0)),
            pl.BlockSpec((1, D), lambda i: (0, 0)),
            pl.BlockSpec((1, 1, BLK), lambda i: (i, 0, 0)),
            pl.BlockSpec((D, D_OUT), lambda i: (0, 0)),
            pl.BlockSpec((1, D_OUT), lambda i: (0, 0)),
        ],
        out_specs=pl.BlockSpec((G, D_OUT), lambda i: (0, 0)),
        out_shape=jax.ShapeDtypeStruct((G, D_OUT), jnp.float32),
        scratch_shapes=[pltpu.VMEM((G, D), jnp.float32)],
    )(q0, q1, s2, d0, d1, b2, batch3, wlin, blin)


def kernel(x, edge_index, batch, W1, b1, W2, b2, Wlin, blin):
    e = edge_index.shape[1]
    n_chunks = -(-e // (NW * CHUNK))
    if n_chunks % 2:
        n_chunks += 1
    e_pad = NW * n_chunks * CHUNK

    pad = jnp.full((e_pad - e,), N_NODES, jnp.int32)
    srcp = jnp.concatenate([edge_index[0], pad]).reshape(NW, n_chunks, CHUNK)
    dstp = jnp.concatenate([edge_index[1], pad]).reshape(NW, n_chunks, CHUNK)

    x_pad = jnp.pad(x, ((0, N_PAD - N_NODES), (0, 0)))
    batch3 = jnp.pad(batch, (0, N_PAD - N_NODES),
                     constant_values=G).reshape(N_PAD // BLK, 1, BLK)
    z16 = jnp.zeros((ROWS_PT, 16), jnp.float32)
    z128 = jnp.zeros((ROWS_PT, D), jnp.float32)
    ones16 = jnp.ones((CHUNK, 16), jnp.float32)

    degp = _sc_degree(dstp, z16, ones16)
    d0, d1 = degp[:N_PAD], degp[N_PAD:]

    xw1 = _tc_matmul(x_pad, W1)
    s1 = _tc_scale(xw1, d0, d1)

    p = _sc_aggregate(s1, srcp, dstp, z128)
    s2 = _tc_mid(p[:N_PAD], p[N_PAD:], s1, d0, d1, b1.reshape(1, D), W2)

    q = _sc_aggregate(s2, srcp, dstp, z128)
    return _tc_final(q[:N_PAD], q[N_PAD:], s2, d0, d1, b2.reshape(1, D),
                     batch3, Wlin, blin.reshape(1, D_OUT))


# speedup vs baseline: 9.3553x; 9.3553x over previous
import functools

import jax
import jax.numpy as jnp
from jax import lax
from jax.experimental import pallas as pl
from jax.experimental.pallas import tpu as pltpu
from jax.experimental.pallas import tpu_sc as plsc

N_NODES = 10000
D = 128
D_OUT = 64
G = 64

NC, NS = 2, 16
NW = NC * NS
CHUNK = 128
N_PAD = 10240
ROWS_PT = N_PAD // NS
BLK = 1024


def _mesh():
    return plsc.VectorSubcoreMesh(core_axis_name="c", subcore_axis_name="s")


def _sc_degree(dstp, z128, ones128):
    half = dstp.shape[1]

    @functools.partial(
        pl.kernel,
        out_type=jax.ShapeDtypeStruct((NC * N_PAD, D), jnp.float32),
        mesh=_mesh(),
        scratch_types=[
            pltpu.VMEM((half, CHUNK), jnp.int32),
            pltpu.VMEM((CHUNK, D), jnp.float32),
            pltpu.VMEM_SHARED((N_PAD, D), jnp.float32),
            pltpu.SemaphoreType.DMA,
        ],
    )
    def deg_kernel(dstp_hbm, z_hbm, ones_hbm, out_hbm, dst_v, ones_v, acc_sh, sem):
        c = lax.axis_index("c")
        s = lax.axis_index("s")
        wid = c * NS + s
        pltpu.sync_copy(ones_hbm, ones_v)
        pltpu.sync_copy(z_hbm, acc_sh.at[pl.ds(s * ROWS_PT, ROWS_PT)])
        plsc.subcore_barrier()

        for h in range(2):
            pltpu.sync_copy(dstp_hbm.at[2 * wid + h], dst_v)

            @pl.loop(0, half)
            def _(j):
                pltpu.sync_copy(ones_v, acc_sh.at[dst_v.at[j]], add=True)

        plsc.subcore_barrier()
        pltpu.sync_copy(
            acc_sh.at[pl.ds(s * ROWS_PT, ROWS_PT)],
            out_hbm.at[pl.ds(c * N_PAD + s * ROWS_PT, ROWS_PT)],
        )

    return deg_kernel(dstp, z128, ones128)


def _sc_aggregate(sfeat, srcp, dstp, z128):
    half = srcp.shape[1]

    @functools.partial(
        pl.kernel,
        out_type=jax.ShapeDtypeStruct((NC * N_PAD, D), jnp.float32),
        mesh=_mesh(),
        scratch_types=[
            pltpu.VMEM((half, CHUNK), jnp.int32),
            pltpu.VMEM((half, CHUNK), jnp.int32),
            pltpu.VMEM((CHUNK, D), jnp.float32),
            pltpu.VMEM((CHUNK, D), jnp.float32),
            pltpu.VMEM_SHARED((N_PAD, D), jnp.float32),
            pltpu.SemaphoreType.DMA,
            pltpu.SemaphoreType.DMA,
        ],
    )
    def agg_kernel(s_hbm, srcp_hbm, dstp_hbm, z_hbm, out_hbm,
                   src_v, dst_v, rows_a, rows_b, acc_sh, sem_a, sem_b):
        c = lax.axis_index("c")
        s = lax.axis_index("s")
        wid = c * NS + s
        pltpu.sync_copy(z_hbm, acc_sh.at[pl.ds(s * ROWS_PT, ROWS_PT)])
        plsc.subcore_barrier()

        for h in range(2):
            pltpu.sync_copy(srcp_hbm.at[2 * wid + h], src_v)
            pltpu.sync_copy(dstp_hbm.at[2 * wid + h], dst_v)

            pltpu.async_copy(s_hbm.at[src_v.at[0]], rows_a, sem_a)

            @pl.loop(0, half, step=2)
            def _(j):
                pltpu.async_copy(s_hbm.at[src_v.at[j + 1]], rows_b, sem_b)
                pltpu.make_async_copy(s_hbm.at[src_v.at[j]], rows_a, sem_a).wait()
                pltpu.sync_copy(rows_a, acc_sh.at[dst_v.at[j]], add=True)

                @pl.when(j + 2 < half)
                def _():
                    pltpu.async_copy(s_hbm.at[src_v.at[j + 2]], rows_a, sem_a)

                pltpu.make_async_copy(s_hbm.at[src_v.at[j + 1]], rows_b, sem_b).wait()
                pltpu.sync_copy(rows_b, acc_sh.at[dst_v.at[j + 1]], add=True)

        plsc.subcore_barrier()
        pltpu.sync_copy(
            acc_sh.at[pl.ds(s * ROWS_PT, ROWS_PT)],
            out_hbm.at[pl.ds(c * N_PAD + s * ROWS_PT, ROWS_PT)],
        )

    return agg_kernel(sfeat, srcp, dstp, z128)


def _tc_matmul(x, w):
    def body(x_ref, w_ref, o_ref):
        o_ref[...] = jnp.dot(x_ref[...], w_ref[...],
                             preferred_element_type=jnp.float32)

    return pl.pallas_call(
        body,
        grid=(N_PAD // BLK,),
        in_specs=[
            pl.BlockSpec((BLK, D), lambda i: (i, 0)),
            pl.BlockSpec((D, D), lambda i: (0, 0)),
        ],
        out_specs=pl.BlockSpec((BLK, D), lambda i: (i, 0)),
        out_shape=jax.ShapeDtypeStruct((N_PAD, D), jnp.float32),
    )(x, w)


def _dis(d0_ref, d1_ref):
    return lax.rsqrt(d0_ref[:, :1] + d1_ref[:, :1] + 1.0)


def _tc_scale(xw, d0, d1):
    def body(xw_ref, d0_ref, d1_ref, o_ref):
        o_ref[...] = xw_ref[...] * _dis(d0_ref, d1_ref)

    return pl.pallas_call(
        body,
        grid=(N_PAD // BLK,),
        in_specs=[
            pl.BlockSpec((BLK, D), lambda i: (i, 0)),
            pl.BlockSpec((BLK, D), lambda i: (i, 0)),
            pl.BlockSpec((BLK, D), lambda i: (i, 0)),
        ],
        out_specs=pl.BlockSpec((BLK, D), lambda i: (i, 0)),
        out_shape=jax.ShapeDtypeStruct((N_PAD, D), jnp.float32),
    )(xw, d0, d1)


def _tc_mid(a0, a1, s1, d0, d1, b1, w2):
    def body(a0_ref, a1_ref, s1_ref, d0_ref, d1_ref, b1_ref, w2_ref, o_ref):
        dis = _dis(d0_ref, d1_ref)
        h = jnp.maximum((a0_ref[...] + a1_ref[...] + s1_ref[...]) * dis
                        + b1_ref[...], 0.0)
        o_ref[...] = jnp.dot(h, w2_ref[...],
                             preferred_element_type=jnp.float32) * dis

    return pl.pallas_call(
        body,
        grid=(N_PAD // BLK,),
        in_specs=[
            pl.BlockSpec((BLK, D), lambda i: (i, 0)),
            pl.BlockSpec((BLK, D), lambda i: (i, 0)),
            pl.BlockSpec((BLK, D), lambda i: (i, 0)),
            pl.BlockSpec((BLK, D), lambda i: (i, 0)),
            pl.BlockSpec((BLK, D), lambda i: (i, 0)),
            pl.BlockSpec((1, D), lambda i: (0, 0)),
            pl.BlockSpec((D, D), lambda i: (0, 0)),
        ],
        out_specs=pl.BlockSpec((BLK, D), lambda i: (i, 0)),
        out_shape=jax.ShapeDtypeStruct((N_PAD, D), jnp.float32),
    )(a0, a1, s1, d0, d1, b1, w2)


def _tc_final(q0, q1, s2, d0, d1, b2, batch3, wlin, blin):
    def body(q0_ref, q1_ref, s2_ref, d0_ref, d1_ref, b2_ref, bt_ref,
             wl_ref, bl_ref, o_ref, acc):
        i = pl.program_id(0)
        dis = _dis(d0_ref, d1_ref)
        h2 = jnp.maximum((q0_ref[...] + q1_ref[...] + s2_ref[...]) * dis
                         + b2_ref[...], 0.0)
        bcol = bt_ref[...].reshape(BLK, 1)
        onehot = (bcol == lax.broadcasted_iota(jnp.int32, (BLK, G), 1)
                  ).astype(jnp.float32)
        contrib = lax.dot_general(onehot, h2, (((0,), (0,)), ((), ())),
                                  preferred_element_type=jnp.float32)

        @pl.when(i == 0)
        def _():
            acc[...] = jnp.zeros_like(acc)

        acc[...] += contrib

        @pl.when(i == pl.num_programs(0) - 1)
        def _():
            o_ref[...] = jnp.dot(acc[...], wl_ref[...],
                                 preferred_element_type=jnp.float32) + bl_ref[...]

    return pl.pallas_call(
        body,
        grid=(N_PAD // BLK,),
        in_specs=[
            pl.BlockSpec((BLK, D), lambda i: (i, 0)),
            pl.BlockSpec((BLK, D), lambda i: (i, 0)),
            pl.BlockSpec((BLK, D), lambda i: (i, 0)),
            pl.BlockSpec((BLK, D), lambda i: (i, 0)),
            pl.BlockSpec((BLK, D), lambda i: (i, 0)),
            pl.BlockSpec((1, D), lambda i: (0, 0)),
            pl.BlockSpec((1, 1, BLK), lambda i: (i, 0, 0)),
            pl.BlockSpec((D, D_OUT), lambda i: (0, 0)),
            pl.BlockSpec((1, D_OUT), lambda i: (0, 0)),
        ],
        out_specs=pl.BlockSpec((G, D_OUT), lambda i: (0, 0)),
        out_shape=jax.ShapeDtypeStruct((G, D_OUT), jnp.float32),
        scratch_shapes=[pltpu.VMEM((G, D), jnp.float32)],
    )(q0, q1, s2, d0, d1, b2, batch3, wlin, blin)


def kernel(x, edge_index, batch, W1, b1, W2, b2, Wlin, blin):
    e = edge_index.shape[1]
    n_chunks = -(-e // (NW * CHUNK))
    n_chunks += -n_chunks % 4
    half = n_chunks // 2
    e_pad = NW * n_chunks * CHUNK

    pad = jnp.full((e_pad - e,), N_NODES, jnp.int32)
    srcp = jnp.concatenate([edge_index[0], pad]).reshape(NW * 2, half, CHUNK)
    dstp = jnp.concatenate([edge_index[1], pad]).reshape(NW * 2, half, CHUNK)

    x_pad = jnp.pad(x, ((0, N_PAD - N_NODES), (0, 0)))
    batch3 = jnp.pad(batch, (0, N_PAD - N_NODES),
                     constant_values=G).reshape(N_PAD // BLK, 1, BLK)
    z128 = jnp.zeros((ROWS_PT, D), jnp.float32)
    ones128 = jnp.ones((CHUNK, D), jnp.float32)

    degp = _sc_degree(dstp, z128, ones128)
    d0, d1 = degp[:N_PAD], degp[N_PAD:]

    xw1 = _tc_matmul(x_pad, W1)
    s1 = _tc_scale(xw1, d0, d1)

    p = _sc_aggregate(s1, srcp, dstp, z128)
    s2 = _tc_mid(p[:N_PAD], p[N_PAD:], s1, d0, d1, b1.reshape(1, D), W2)

    q = _sc_aggregate(s2, srcp, dstp, z128)
    return _tc_final(q[:N_PAD], q[N_PAD:], s2, d0, d1, b2.reshape(1, D),
                     batch3, Wlin, blin.reshape(1, D_OUT))


# speedup vs baseline: 25.2473x; 2.6987x over previous
import functools

import jax
import jax.numpy as jnp
from jax import lax
from jax.experimental import pallas as pl
from jax.experimental.pallas import tpu as pltpu
from jax.experimental.pallas import tpu_sc as plsc

N_NODES = 10000
D = 128
D_OUT = 64
G = 64

NC, NS = 2, 16
NW = NC * NS
CHUNK = 128
N_PAD = 10240
ROWS_PT = N_PAD // NS
BLK = 1024


def _mesh():
    return plsc.VectorSubcoreMesh(core_axis_name="c", subcore_axis_name="s")


def _sc_degree(dstp, z128, ones128):
    half = dstp.shape[1]

    @functools.partial(
        pl.kernel,
        out_type=jax.ShapeDtypeStruct((NC * N_PAD, D), jnp.float32),
        mesh=_mesh(),
        scratch_types=[
            pltpu.VMEM((half, CHUNK), jnp.int32),
            pltpu.VMEM((CHUNK, D), jnp.float32),
            pltpu.VMEM_SHARED((N_PAD, D), jnp.float32),
            pltpu.SemaphoreType.DMA,
        ],
    )
    def deg_kernel(dstp_hbm, z_hbm, ones_hbm, out_hbm, dst_v, ones_v, acc_sh, sem):
        c = lax.axis_index("c")
        s = lax.axis_index("s")
        wid = c * NS + s
        pltpu.sync_copy(ones_hbm, ones_v)
        pltpu.sync_copy(z_hbm, acc_sh.at[pl.ds(s * ROWS_PT, ROWS_PT)])
        plsc.subcore_barrier()

        for h in range(2):
            pltpu.sync_copy(dstp_hbm.at[2 * wid + h], dst_v)

            @pl.loop(0, half)
            def _(j):
                pltpu.sync_copy(ones_v, acc_sh.at[dst_v.at[j]], add=True)

        plsc.subcore_barrier()
        pltpu.sync_copy(
            acc_sh.at[pl.ds(s * ROWS_PT, ROWS_PT)],
            out_hbm.at[pl.ds(c * N_PAD + s * ROWS_PT, ROWS_PT)],
        )

    return deg_kernel(dstp, z128, ones128)


def _sc_aggregate(sfeat, srcp, dstp, z128):
    half = srcp.shape[1]

    @functools.partial(
        pl.kernel,
        out_type=jax.ShapeDtypeStruct((NC * N_PAD, D), jnp.float32),
        mesh=_mesh(),
        scratch_types=[
            pltpu.VMEM((half, CHUNK), jnp.int32),
            pltpu.VMEM((half, CHUNK), jnp.int32),
            pltpu.VMEM((CHUNK, D), jnp.float32),
            pltpu.VMEM((CHUNK, D), jnp.float32),
            pltpu.VMEM_SHARED((N_PAD, D), jnp.float32),
            pltpu.SemaphoreType.DMA,
            pltpu.SemaphoreType.DMA,
        ],
    )
    def agg_kernel(s_hbm, srcp_hbm, dstp_hbm, z_hbm, out_hbm,
                   src_v, dst_v, rows_a, rows_b, acc_sh, sem_a, sem_b):
        c = lax.axis_index("c")
        s = lax.axis_index("s")
        wid = c * NS + s
        pltpu.sync_copy(z_hbm, acc_sh.at[pl.ds(s * ROWS_PT, ROWS_PT)])
        plsc.subcore_barrier()

        for h in range(2):
            pltpu.sync_copy(srcp_hbm.at[2 * wid + h], src_v)
            pltpu.sync_copy(dstp_hbm.at[2 * wid + h], dst_v)

            pltpu.async_copy(s_hbm.at[src_v.at[0]], rows_a, sem_a)

            @pl.loop(0, half, step=2)
            def _(j):
                pltpu.async_copy(s_hbm.at[src_v.at[j + 1]], rows_b, sem_b)
                pltpu.make_async_copy(s_hbm.at[src_v.at[j]], rows_a, sem_a).wait()
                pltpu.sync_copy(rows_a, acc_sh.at[dst_v.at[j]], add=True)

                @pl.when(j + 2 < half)
                def _():
                    pltpu.async_copy(s_hbm.at[src_v.at[j + 2]], rows_a, sem_a)

                pltpu.make_async_copy(s_hbm.at[src_v.at[j + 1]], rows_b, sem_b).wait()
                pltpu.sync_copy(rows_b, acc_sh.at[dst_v.at[j + 1]], add=True)

        plsc.subcore_barrier()
        pltpu.sync_copy(
            acc_sh.at[pl.ds(s * ROWS_PT, ROWS_PT)],
            out_hbm.at[pl.ds(c * N_PAD + s * ROWS_PT, ROWS_PT)],
        )

    return agg_kernel(sfeat, srcp, dstp, z128)


def _tc_matmul(x, w):
    def body(x_ref, w_ref, o_ref):
        o_ref[...] = jnp.dot(x_ref[...], w_ref[...],
                             preferred_element_type=jnp.float32)

    return pl.pallas_call(
        body,
        grid=(N_PAD // BLK,),
        in_specs=[
            pl.BlockSpec((BLK, D), lambda i: (i, 0)),
            pl.BlockSpec((D, D), lambda i: (0, 0)),
        ],
        out_specs=pl.BlockSpec((BLK, D), lambda i: (i, 0)),
        out_shape=jax.ShapeDtypeStruct((N_PAD, D), jnp.float32),
    )(x, w)


def _dis(d0_ref, d1_ref):
    return lax.rsqrt(d0_ref[:, :1] + d1_ref[:, :1] + 1.0)


def _tc_scale(xw, d0, d1):
    def body(xw_ref, d0_ref, d1_ref, o_ref):
        o_ref[...] = xw_ref[...] * _dis(d0_ref, d1_ref)

    return pl.pallas_call(
        body,
        grid=(N_PAD // BLK,),
        in_specs=[
            pl.BlockSpec((BLK, D), lambda i: (i, 0)),
            pl.BlockSpec((BLK, D), lambda i: (i, 0)),
            pl.BlockSpec((BLK, D), lambda i: (i, 0)),
        ],
        out_specs=pl.BlockSpec((BLK, D), lambda i: (i, 0)),
        out_shape=jax.ShapeDtypeStruct((N_PAD, D), jnp.float32),
    )(xw, d0, d1)


def _tc_mid(a0, a1, s1, d0, d1, b1, w2):
    def body(a0_ref, a1_ref, s1_ref, d0_ref, d1_ref, b1_ref, w2_ref, o_ref):
        dis = _dis(d0_ref, d1_ref)
        h = jnp.maximum((a0_ref[...] + a1_ref[...] + s1_ref[...]) * dis
                        + b1_ref[...], 0.0)
        o_ref[...] = jnp.dot(h, w2_ref[...],
                             preferred_element_type=jnp.float32) * dis

    return pl.pallas_call(
        body,
        grid=(N_PAD // BLK,),
        in_specs=[
            pl.BlockSpec((BLK, D), lambda i: (i, 0)),
            pl.BlockSpec((BLK, D), lambda i: (i, 0)),
            pl.BlockSpec((BLK, D), lambda i: (i, 0)),
            pl.BlockSpec((BLK, D), lambda i: (i, 0)),
            pl.BlockSpec((BLK, D), lambda i: (i, 0)),
            pl.BlockSpec((1, D), lambda i: (0, 0)),
            pl.BlockSpec((D, D), lambda i: (0, 0)),
        ],
        out_specs=pl.BlockSpec((BLK, D), lambda i: (i, 0)),
        out_shape=jax.ShapeDtypeStruct((N_PAD, D), jnp.float32),
    )(a0, a1, s1, d0, d1, b1, w2)


def _tc_final(q0, q1, s2, d0, d1, b2, batch3, wlin, blin):
    def body(q0_ref, q1_ref, s2_ref, d0_ref, d1_ref, b2_ref, bt_ref,
             wl_ref, bl_ref, o_ref, acc):
        i = pl.program_id(0)
        dis = _dis(d0_ref, d1_ref)
        h2 = jnp.maximum((q0_ref[...] + q1_ref[...] + s2_ref[...]) * dis
                         + b2_ref[...], 0.0)
        bcol = bt_ref[...].reshape(BLK, 1)
        onehot = (bcol == lax.broadcasted_iota(jnp.int32, (BLK, G), 1)
                  ).astype(jnp.float32)
        contrib = lax.dot_general(onehot, h2, (((0,), (0,)), ((), ())),
                                  preferred_element_type=jnp.float32)

        @pl.when(i == 0)
        def _():
            acc[...] = jnp.zeros_like(acc)

        acc[...] += contrib

        @pl.when(i == pl.num_programs(0) - 1)
        def _():
            o_ref[...] = jnp.dot(acc[...], wl_ref[...],
                                 preferred_element_type=jnp.float32) + bl_ref[...]

    return pl.pallas_call(
        body,
        grid=(N_PAD // BLK,),
        in_specs=[
            pl.BlockSpec((BLK, D), lambda i: (i, 0)),
            pl.BlockSpec((BLK, D), lambda i: (i, 0)),
            pl.BlockSpec((BLK, D), lambda i: (i, 0)),
            pl.BlockSpec((BLK, D), lambda i: (i, 0)),
            pl.BlockSpec((BLK, D), lambda i: (i, 0)),
            pl.BlockSpec((1, D), lambda i: (0, 0)),
            pl.BlockSpec((1, 1, BLK), lambda i: (i, 0, 0)),
            pl.BlockSpec((D, D_OUT), lambda i: (0, 0)),
            pl.BlockSpec((1, D_OUT), lambda i: (0, 0)),
        ],
        out_specs=pl.BlockSpec((G, D_OUT), lambda i: (0, 0)),
        out_shape=jax.ShapeDtypeStruct((G, D_OUT), jnp.float32),
        scratch_shapes=[pltpu.VMEM((G, D), jnp.float32)],
    )(q0, q1, s2, d0, d1, b2, batch3, wlin, blin)


def kernel(x, edge_index, batch, W1, b1, W2, b2, Wlin, blin):
    e = edge_index.shape[1]
    n_chunks = -(-e // (NW * CHUNK))
    n_chunks += -n_chunks % 4
    half = n_chunks // 2
    e_pad = NW * n_chunks * CHUNK

    pad = N_NODES + jnp.arange(e_pad - e, dtype=jnp.int32) % (N_PAD - N_NODES)
    srcp = jnp.concatenate([edge_index[0], pad]).reshape(NW * 2, half, CHUNK)
    dstp = jnp.concatenate([edge_index[1], pad]).reshape(NW * 2, half, CHUNK)

    x_pad = jnp.pad(x, ((0, N_PAD - N_NODES), (0, 0)))
    batch3 = jnp.pad(batch, (0, N_PAD - N_NODES),
                     constant_values=G).reshape(N_PAD // BLK, 1, BLK)
    z128 = jnp.zeros((ROWS_PT, D), jnp.float32)
    ones128 = jnp.ones((CHUNK, D), jnp.float32)

    degp = _sc_degree(dstp, z128, ones128)
    d0, d1 = degp[:N_PAD], degp[N_PAD:]

    xw1 = _tc_matmul(x_pad, W1)
    s1 = _tc_scale(xw1, d0, d1)

    p = _sc_aggregate(s1, srcp, dstp, z128)
    s2 = _tc_mid(p[:N_PAD], p[N_PAD:], s1, d0, d1, b1.reshape(1, D), W2)

    q = _sc_aggregate(s2, srcp, dstp, z128)
    return _tc_final(q[:N_PAD], q[N_PAD:], s2, d0, d1, b2.reshape(1, D),
                     batch3, Wlin, blin.reshape(1, D_OUT))


# speedup vs baseline: 27.2508x; 1.0794x over previous
import functools

import jax
import jax.numpy as jnp
from jax import lax
from jax.experimental import pallas as pl
from jax.experimental.pallas import tpu as pltpu
from jax.experimental.pallas import tpu_sc as plsc

N_NODES = 10000
D = 128
D_OUT = 64
G = 64

NC, NS = 2, 16
NW = NC * NS
SUB = 64
NBUF = 4
N_PAD = 10240
ROWS_PT = N_PAD // NS
BLK = 1024
NBLK = N_PAD // BLK


def _mesh():
    return plsc.VectorSubcoreMesh(core_axis_name="c", subcore_axis_name="s")


def _sc_degree(dstp, z128, ones128):
    qs = dstp.shape[1]

    @functools.partial(
        pl.kernel,
        out_type=jax.ShapeDtypeStruct((NC * N_PAD, D), jnp.float32),
        mesh=_mesh(),
        scratch_types=[
            pltpu.VMEM((qs, SUB), jnp.int32),
            pltpu.VMEM((SUB, D), jnp.float32),
            pltpu.VMEM_SHARED((N_PAD, D), jnp.float32),
            pltpu.SemaphoreType.DMA,
        ],
    )
    def deg_kernel(dstp_hbm, z_hbm, ones_hbm, out_hbm, dst_v, ones_v, acc_sh, sem):
        c = lax.axis_index("c")
        s = lax.axis_index("s")
        wid = c * NS + s
        pltpu.sync_copy(ones_hbm, ones_v)
        pltpu.sync_copy(z_hbm, acc_sh.at[pl.ds(s * ROWS_PT, ROWS_PT)])
        plsc.subcore_barrier()

        for h in range(4):
            pltpu.sync_copy(dstp_hbm.at[4 * wid + h], dst_v)

            @pl.loop(0, qs)
            def _(j):
                pltpu.sync_copy(ones_v, acc_sh.at[dst_v.at[j]], add=True)

        plsc.subcore_barrier()
        pltpu.sync_copy(
            acc_sh.at[pl.ds(s * ROWS_PT, ROWS_PT)],
            out_hbm.at[pl.ds(c * N_PAD + s * ROWS_PT, ROWS_PT)],
        )

    return deg_kernel(dstp, z128, ones128)


def _sc_aggregate(sfeat, srcp, dstp, z128):
    qs = srcp.shape[1]

    @functools.partial(
        pl.kernel,
        out_type=jax.ShapeDtypeStruct((NC * N_PAD, D), jnp.float32),
        mesh=_mesh(),
        scratch_types=[
            pltpu.VMEM((qs, SUB), jnp.int32),
            pltpu.VMEM((qs, SUB), jnp.int32),
        ] + [pltpu.VMEM((SUB, D), jnp.float32) for _ in range(NBUF)] + [
            pltpu.VMEM_SHARED((N_PAD, D), jnp.float32),
        ] + [pltpu.SemaphoreType.DMA for _ in range(NBUF)],
    )
    def agg_kernel(s_hbm, srcp_hbm, dstp_hbm, z_hbm, out_hbm,
                   src_v, dst_v, *rows_acc_sems):
        rows = rows_acc_sems[:NBUF]
        acc_sh = rows_acc_sems[NBUF]
        sems = rows_acc_sems[NBUF + 1:]
        c = lax.axis_index("c")
        s = lax.axis_index("s")
        wid = c * NS + s
        pltpu.sync_copy(z_hbm, acc_sh.at[pl.ds(s * ROWS_PT, ROWS_PT)])
        plsc.subcore_barrier()

        for h in range(4):
            pltpu.sync_copy(srcp_hbm.at[4 * wid + h], src_v)
            pltpu.sync_copy(dstp_hbm.at[4 * wid + h], dst_v)
            for b in range(NBUF):
                pltpu.async_copy(s_hbm.at[src_v.at[b]], rows[b], sems[b])

            @pl.loop(0, qs, step=NBUF)
            def _(j):
                for b in range(NBUF):
                    pltpu.make_async_copy(
                        s_hbm.at[src_v.at[j + b]], rows[b], sems[b]).wait()
                    pltpu.sync_copy(rows[b], acc_sh.at[dst_v.at[j + b]], add=True)

                    @pl.when(j + b + NBUF < qs)
                    def _():
                        pltpu.async_copy(
                            s_hbm.at[src_v.at[j + b + NBUF]], rows[b], sems[b])

        plsc.subcore_barrier()
        pltpu.sync_copy(
            acc_sh.at[pl.ds(s * ROWS_PT, ROWS_PT)],
            out_hbm.at[pl.ds(c * N_PAD + s * ROWS_PT, ROWS_PT)],
        )

    return agg_kernel(sfeat, srcp, dstp, z128)


def _part_specs():
    return [
        pl.BlockSpec((BLK, D), lambda i: (i, 0)),
        pl.BlockSpec((BLK, D), lambda i: (i + NBLK, 0)),
    ]


def _tc_matmul(x, w):
    def body(x_ref, w_ref, o_ref):
        o_ref[...] = jnp.dot(x_ref[...], w_ref[...],
                             preferred_element_type=jnp.float32)

    return pl.pallas_call(
        body,
        grid=(NBLK,),
        in_specs=[
            pl.BlockSpec((BLK, D), lambda i: (i, 0)),
            pl.BlockSpec((D, D), lambda i: (0, 0)),
        ],
        out_specs=pl.BlockSpec((BLK, D), lambda i: (i, 0)),
        out_shape=jax.ShapeDtypeStruct((N_PAD, D), jnp.float32),
    )(x, w)


def _dis(d0_ref, d1_ref):
    return lax.rsqrt(d0_ref[:, :1] + d1_ref[:, :1] + 1.0)


def _tc_scale(xw, degp):
    def body(xw_ref, d0_ref, d1_ref, o_ref):
        o_ref[...] = xw_ref[...] * _dis(d0_ref, d1_ref)

    return pl.pallas_call(
        body,
        grid=(NBLK,),
        in_specs=[pl.BlockSpec((BLK, D), lambda i: (i, 0))] + _part_specs(),
        out_specs=pl.BlockSpec((BLK, D), lambda i: (i, 0)),
        out_shape=jax.ShapeDtypeStruct((N_PAD, D), jnp.float32),
    )(xw, degp, degp)


def _tc_mid(p, s1, degp, b1, w2):
    def body(a0_ref, a1_ref, s1_ref, d0_ref, d1_ref, b1_ref, w2_ref, o_ref):
        dis = _dis(d0_ref, d1_ref)
        h = jnp.maximum((a0_ref[...] + a1_ref[...] + s1_ref[...]) * dis
                        + b1_ref[...], 0.0)
        o_ref[...] = jnp.dot(h, w2_ref[...],
                             preferred_element_type=jnp.float32) * dis

    return pl.pallas_call(
        body,
        grid=(NBLK,),
        in_specs=_part_specs() + [
            pl.BlockSpec((BLK, D), lambda i: (i, 0)),
        ] + _part_specs() + [
            pl.BlockSpec((1, D), lambda i: (0, 0)),
            pl.BlockSpec((D, D), lambda i: (0, 0)),
        ],
        out_specs=pl.BlockSpec((BLK, D), lambda i: (i, 0)),
        out_shape=jax.ShapeDtypeStruct((N_PAD, D), jnp.float32),
    )(p, p, s1, degp, degp, b1, w2)


def _tc_final(q, s2, degp, b2, batch3, wlin, blin):
    def body(q0_ref, q1_ref, s2_ref, d0_ref, d1_ref, b2_ref, bt_ref,
             wl_ref, bl_ref, o_ref, acc):
        i = pl.program_id(0)
        dis = _dis(d0_ref, d1_ref)
        h2 = jnp.maximum((q0_ref[...] + q1_ref[...] + s2_ref[...]) * dis
                         + b2_ref[...], 0.0)
        bcol = bt_ref[...].reshape(BLK, 1)
        onehot = (bcol == lax.broadcasted_iota(jnp.int32, (BLK, G), 1)
                  ).astype(jnp.float32)
        contrib = lax.dot_general(onehot, h2, (((0,), (0,)), ((), ())),
                                  preferred_element_type=jnp.float32)

        @pl.when(i == 0)
        def _():
            acc[...] = jnp.zeros_like(acc)

        acc[...] += contrib

        @pl.when(i == pl.num_programs(0) - 1)
        def _():
            o_ref[...] = jnp.dot(acc[...], wl_ref[...],
                                 preferred_element_type=jnp.float32) + bl_ref[...]

    return pl.pallas_call(
        body,
        grid=(NBLK,),
        in_specs=_part_specs() + [
            pl.BlockSpec((BLK, D), lambda i: (i, 0)),
        ] + _part_specs() + [
            pl.BlockSpec((1, D), lambda i: (0, 0)),
            pl.BlockSpec((1, 1, BLK), lambda i: (i, 0, 0)),
            pl.BlockSpec((D, D_OUT), lambda i: (0, 0)),
            pl.BlockSpec((1, D_OUT), lambda i: (0, 0)),
        ],
        out_specs=pl.BlockSpec((G, D_OUT), lambda i: (0, 0)),
        out_shape=jax.ShapeDtypeStruct((G, D_OUT), jnp.float32),
        scratch_shapes=[pltpu.VMEM((G, D), jnp.float32)],
    )(q, q, s2, degp, degp, b2, batch3, wlin, blin)


def kernel(x, edge_index, batch, W1, b1, W2, b2, Wlin, blin):
    e = edge_index.shape[1]
    nsub = -(-e // (NW * SUB))
    nsub += -nsub % (4 * NBUF)
    qs = nsub // 4
    e_pad = NW * nsub * SUB

    pad = N_NODES + jnp.arange(e_pad - e, dtype=jnp.int32) % (N_PAD - N_NODES)
    srcp = jnp.concatenate([edge_index[0], pad]).reshape(NW * 4, qs, SUB)
    dstp = jnp.concatenate([edge_index[1], pad]).reshape(NW * 4, qs, SUB)

    x_pad = jnp.pad(x, ((0, N_PAD - N_NODES), (0, 0)))
    batch3 = jnp.pad(batch, (0, N_PAD - N_NODES),
                     constant_values=G).reshape(NBLK, 1, BLK)
    z128 = jnp.zeros((ROWS_PT, D), jnp.float32)
    ones128 = jnp.ones((SUB, D), jnp.float32)

    degp = _sc_degree(dstp, z128, ones128)

    xw1 = _tc_matmul(x_pad, W1)
    s1 = _tc_scale(xw1, degp)

    p = _sc_aggregate(s1, srcp, dstp, z128)
    s2 = _tc_mid(p, s1, degp, b1.reshape(1, D), W2)

    q = _sc_aggregate(s2, srcp, dstp, z128)
    return _tc_final(q, s2, degp, b2.reshape(1, D),
                     batch3, Wlin, blin.reshape(1, D_OUT))


# speedup vs baseline: 27.3442x; 1.0034x over previous
import functools

import jax
import jax.numpy as jnp
from jax import lax
from jax.experimental import pallas as pl
from jax.experimental.pallas import tpu as pltpu
from jax.experimental.pallas import tpu_sc as plsc

N_NODES = 10000
D = 128
D_OUT = 64
G = 64

NC, NS = 2, 16
NW = NC * NS
SUB = 64
NBUF = 4
N_PAD = 10240
ROWS_PT = N_PAD // NS
BLK = 1024
NBLK = N_PAD // BLK


def _mesh():
    return plsc.VectorSubcoreMesh(core_axis_name="c", subcore_axis_name="s")


def _sc_degree(dstp, z128, ones128):
    qs = dstp.shape[1]

    @functools.partial(
        pl.kernel,
        out_type=jax.ShapeDtypeStruct((NC * N_PAD, D), jnp.float32),
        mesh=_mesh(),
        scratch_types=[
            pltpu.VMEM((qs, SUB), jnp.int32),
            pltpu.VMEM((SUB, D), jnp.float32),
            pltpu.VMEM_SHARED((N_PAD, D), jnp.float32),
            pltpu.SemaphoreType.DMA,
        ],
    )
    def deg_kernel(dstp_hbm, z_hbm, ones_hbm, out_hbm, dst_v, ones_v, acc_sh, sem):
        c = lax.axis_index("c")
        s = lax.axis_index("s")
        wid = c * NS + s
        pltpu.sync_copy(ones_hbm, ones_v)
        pltpu.sync_copy(z_hbm, acc_sh.at[pl.ds(s * ROWS_PT, ROWS_PT)])
        plsc.subcore_barrier()

        for h in range(4):
            pltpu.sync_copy(dstp_hbm.at[4 * wid + h], dst_v)

            @pl.loop(0, qs)
            def _(j):
                pltpu.sync_copy(ones_v, acc_sh.at[dst_v.at[j]], add=True)

        plsc.subcore_barrier()
        pltpu.sync_copy(
            acc_sh.at[pl.ds(s * ROWS_PT, ROWS_PT)],
            out_hbm.at[pl.ds(c * N_PAD + s * ROWS_PT, ROWS_PT)],
        )

    return deg_kernel(dstp, z128, ones128)


def _sc_aggregate(sfeat, srcp, dstp, z128):
    qs = srcp.shape[1]

    @functools.partial(
        pl.kernel,
        out_type=jax.ShapeDtypeStruct((NC * N_PAD, D), jnp.float32),
        mesh=_mesh(),
        scratch_types=[
            pltpu.VMEM((qs, SUB), jnp.int32),
            pltpu.VMEM((qs, SUB), jnp.int32),
        ] + [pltpu.VMEM((SUB, D), jnp.float32) for _ in range(NBUF)] + [
            pltpu.VMEM_SHARED((N_PAD, D), jnp.float32),
        ] + [pltpu.SemaphoreType.DMA for _ in range(NBUF)],
    )
    def agg_kernel(s_hbm, srcp_hbm, dstp_hbm, z_hbm, out_hbm,
                   src_v, dst_v, *rows_acc_sems):
        rows = rows_acc_sems[:NBUF]
        acc_sh = rows_acc_sems[NBUF]
        sems = rows_acc_sems[NBUF + 1:]
        c = lax.axis_index("c")
        s = lax.axis_index("s")
        wid = c * NS + s
        pltpu.sync_copy(z_hbm, acc_sh.at[pl.ds(s * ROWS_PT, ROWS_PT)])
        plsc.subcore_barrier()

        for h in range(4):
            pltpu.sync_copy(srcp_hbm.at[4 * wid + h], src_v)
            pltpu.sync_copy(dstp_hbm.at[4 * wid + h], dst_v)
            for b in range(NBUF):
                pltpu.async_copy(s_hbm.at[src_v.at[b]], rows[b], sems[b])

            @pl.loop(0, qs, step=NBUF)
            def _(j):
                for b in range(NBUF):
                    pltpu.make_async_copy(
                        s_hbm.at[src_v.at[j + b]], rows[b], sems[b]).wait()
                    pltpu.sync_copy(rows[b], acc_sh.at[dst_v.at[j + b]], add=True)

                    @pl.when(j + b + NBUF < qs)
                    def _():
                        pltpu.async_copy(
                            s_hbm.at[src_v.at[j + b + NBUF]], rows[b], sems[b])

        plsc.subcore_barrier()
        pltpu.sync_copy(
            acc_sh.at[pl.ds(s * ROWS_PT, ROWS_PT)],
            out_hbm.at[pl.ds(c * N_PAD + s * ROWS_PT, ROWS_PT)],
        )

    return agg_kernel(sfeat, srcp, dstp, z128)


def _part_specs():
    return [
        pl.BlockSpec((BLK, D), lambda i: (i, 0)),
        pl.BlockSpec((BLK, D), lambda i: (i + NBLK, 0)),
    ]


def _tc_matmul(x, w):
    def body(x_ref, w_ref, o_ref):
        o_ref[...] = jnp.dot(x_ref[...], w_ref[...],
                             preferred_element_type=jnp.float32)

    return pl.pallas_call(
        body,
        grid=(NBLK,),
        in_specs=[
            pl.BlockSpec((BLK, D), lambda i: (i, 0)),
            pl.BlockSpec((D, D), lambda i: (0, 0)),
        ],
        out_specs=pl.BlockSpec((BLK, D), lambda i: (i, 0)),
        out_shape=jax.ShapeDtypeStruct((N_PAD, D), jnp.float32),
    )(x, w)


def _dis(d0_ref, d1_ref):
    return lax.rsqrt(d0_ref[:, :1] + d1_ref[:, :1] + 1.0)


def _tc_scale(xw, degp):
    def body(xw_ref, d0_ref, d1_ref, o_ref):
        o_ref[...] = xw_ref[...] * _dis(d0_ref, d1_ref)

    return pl.pallas_call(
        body,
        grid=(NBLK,),
        in_specs=[pl.BlockSpec((BLK, D), lambda i: (i, 0))] + _part_specs(),
        out_specs=pl.BlockSpec((BLK, D), lambda i: (i, 0)),
        out_shape=jax.ShapeDtypeStruct((N_PAD, D), jnp.float32),
    )(xw, degp, degp)


def _tc_mid(p, s1, degp, b1, w2):
    def body(a0_ref, a1_ref, s1_ref, d0_ref, d1_ref, b1_ref, w2_ref, o_ref):
        dis = _dis(d0_ref, d1_ref)
        h = jnp.maximum((a0_ref[...] + a1_ref[...] + s1_ref[...]) * dis
                        + b1_ref[...], 0.0)
        o_ref[...] = jnp.dot(h, w2_ref[...],
                             preferred_element_type=jnp.float32) * dis

    return pl.pallas_call(
        body,
        grid=(NBLK,),
        in_specs=_part_specs() + [
            pl.BlockSpec((BLK, D), lambda i: (i, 0)),
        ] + _part_specs() + [
            pl.BlockSpec((1, D), lambda i: (0, 0)),
            pl.BlockSpec((D, D), lambda i: (0, 0)),
        ],
        out_specs=pl.BlockSpec((BLK, D), lambda i: (i, 0)),
        out_shape=jax.ShapeDtypeStruct((N_PAD, D), jnp.float32),
    )(p, p, s1, degp, degp, b1, w2)


def _tc_final(q, s2, degp, b2, batch3, wlin, blin):
    def body(q0_ref, q1_ref, s2_ref, d0_ref, d1_ref, b2_ref, bt_ref,
             wl_ref, bl_ref, o_ref, acc):
        i = pl.program_id(0)
        dis = _dis(d0_ref, d1_ref)
        h2 = jnp.maximum((q0_ref[...] + q1_ref[...] + s2_ref[...]) * dis
                         + b2_ref[...], 0.0)
        bcol = bt_ref[...].reshape(BLK, 1)
        onehot = (bcol == lax.broadcasted_iota(jnp.int32, (BLK, G), 1)
                  ).astype(jnp.float32)
        contrib = lax.dot_general(onehot, h2, (((0,), (0,)), ((), ())),
                                  preferred_element_type=jnp.float32)

        @pl.when(i == 0)
        def _():
            acc[...] = jnp.zeros_like(acc)

        acc[...] += contrib

        @pl.when(i == pl.num_programs(0) - 1)
        def _():
            o_ref[...] = jnp.dot(acc[...], wl_ref[...],
                                 preferred_element_type=jnp.float32) + bl_ref[...]

    return pl.pallas_call(
        body,
        grid=(NBLK,),
        in_specs=_part_specs() + [
            pl.BlockSpec((BLK, D), lambda i: (i, 0)),
        ] + _part_specs() + [
            pl.BlockSpec((1, D), lambda i: (0, 0)),
            pl.BlockSpec((1, 1, BLK), lambda i: (i, 0, 0)),
            pl.BlockSpec((D, D_OUT), lambda i: (0, 0)),
            pl.BlockSpec((1, D_OUT), lambda i: (0, 0)),
        ],
        out_specs=pl.BlockSpec((G, D_OUT), lambda i: (0, 0)),
        out_shape=jax.ShapeDtypeStruct((G, D_OUT), jnp.float32),
        scratch_shapes=[pltpu.VMEM((G, D), jnp.float32)],
    )(q, q, s2, degp, degp, b2, batch3, wlin, blin)


def kernel(x, edge_index, batch, W1, b1, W2, b2, Wlin, blin):
    e = edge_index.shape[1]
    nsub = -(-e // (NW * SUB))
    nsub += -nsub % (4 * NBUF)
    qs = nsub // 4
    e_pad = NW * nsub * SUB

    spare = N_PAD - N_NODES
    pad_d = N_NODES + jnp.arange(e_pad - e, dtype=jnp.int32) % spare
    pad_s = N_NODES + spare - 1 - jnp.arange(e_pad - e, dtype=jnp.int32) % spare
    dstp = jnp.concatenate([edge_index[1], pad_d]).reshape(NW * 4, qs, SUB)
    srcp = jnp.concatenate([edge_index[0], pad_s]).reshape(NW * 4, qs, SUB)

    x_pad = jnp.pad(x, ((0, N_PAD - N_NODES), (0, 0)))
    batch3 = jnp.pad(batch, (0, N_PAD - N_NODES),
                     constant_values=G).reshape(NBLK, 1, BLK)
    z128 = jnp.zeros((ROWS_PT, D), jnp.float32)
    ones128 = jnp.ones((SUB, D), jnp.float32)

    degp = _sc_degree(dstp, z128, ones128)

    xw1 = _tc_matmul(x_pad, W1)
    s1 = _tc_scale(xw1, degp)

    p = _sc_aggregate(s1, srcp, dstp, z128)
    s2 = _tc_mid(p, s1, degp, b1.reshape(1, D), W2)

    q = _sc_aggregate(s2, srcp, dstp, z128)
    return _tc_final(q, s2, degp, b2.reshape(1, D),
                     batch3, Wlin, blin.reshape(1, D_OUT))
